# baseline (speedup 1.0000x reference)
import functools

import jax
import jax.numpy as jnp
from jax import lax
from jax.experimental import pallas as pl
from jax.experimental.pallas import tpu as pltpu
from jax.experimental.pallas import tpu_sc as plsc

NC = 2
NS = 16
NW = NC * NS

C = 80


def _sc_scatter_parts(ea1, ea2, idx1, idx2, zrows, n_pad):
    e, d = ea1.shape
    ew = e // NW
    t = ew // C
    rows_sub = n_pad // NS
    zc = rows_sub // C

    mesh = plsc.VectorSubcoreMesh(core_axis_name="c", subcore_axis_name="s")

    tp = t // 2

    @functools.partial(
        pl.kernel,
        out_type=jax.ShapeDtypeStruct((NC, n_pad, d), jnp.float32),
        mesh=mesh,
        scratch_types=[
            pltpu.VMEM_SHARED((n_pad, d), jnp.float32),
            [pltpu.VMEM((2 * C, d), jnp.float32)] * 2,
            [pltpu.VMEM((1, C), jnp.int32)] * 4,
            [pltpu.SemaphoreType.DMA] * 2,
            [pltpu.SemaphoreType.DMA] * 4,
            [pltpu.SemaphoreType.DMA] * 4,
        ],
    )
    def sc_kernel(ea1_hbm, ea2_hbm, i1_hbm, i2_hbm, z_hbm, parts_hbm,
                  acc, ebufs, ibufs, sem_e, sem_i, sem_s):
        c = lax.axis_index("c")
        s = lax.axis_index("s")
        wid = c * NS + s
        base_e = wid * ew
        base_i = wid * t
        arrs = ((ea1_hbm, i1_hbm), (ea2_hbm, i2_hbm))

        def start_loads(a, j):
            ea_hbm, ii_hbm = arrs[a]
            pltpu.async_copy(ii_hbm.at[pl.ds(base_i + 2 * j, 1)],
                             ibufs[2 * a], sem_i[2 * a])
            pltpu.async_copy(ii_hbm.at[pl.ds(base_i + 2 * j + 1, 1)],
                             ibufs[2 * a + 1], sem_i[2 * a + 1])
            pltpu.async_copy(ea_hbm.at[pl.ds(base_e + 2 * j * C, 2 * C)],
                             ebufs[a], sem_e[a])

        def wait_loads(a):
            ea_hbm, ii_hbm = arrs[a]
            for h in (0, 1):
                pltpu.make_async_copy(ii_hbm.at[pl.ds(base_i, 1)],
                                      ibufs[2 * a + h], sem_i[2 * a + h]).wait()
            pltpu.make_async_copy(ea_hbm.at[pl.ds(base_e, 2 * C)],
                                  ebufs[a], sem_e[a]).wait()

        def scatter_pair(a):
            for h in (0, 1):
                k = 2 * a + h
                pltpu.async_copy(ebufs[a].at[pl.ds(h * C, C)],
                                 acc.at[ibufs[k].at[0]], sem_s[k], add=True)
            for h in (0, 1):
                k = 2 * a + h
                pltpu.make_async_copy(ebufs[a].at[pl.ds(h * C, C)],
                                      acc.at[ibufs[k].at[0]], sem_s[k]).wait()

        for a in (0, 1):
            start_loads(a, 0)

        @pl.loop(0, zc)
        def _(k):
            pltpu.sync_copy(z_hbm, acc.at[pl.ds(s * rows_sub + k * C, C)])

        plsc.subcore_barrier()

        @pl.loop(0, tp)
        def _(j):
            for a in (0, 1):
                wait_loads(a)
                scatter_pair(a)

                @pl.when(j + 1 < tp)
                def _():
                    start_loads(a, j + 1)

        for a in (0, 1):
            ea_hbm, ii_hbm = arrs[a]
            pltpu.async_copy(ii_hbm.at[pl.ds(base_i + t - 1, 1)],
                             ibufs[2 * a], sem_i[2 * a])
            pltpu.async_copy(ea_hbm.at[pl.ds(base_e + (t - 1) * C, C)],
                             ebufs[a].at[pl.ds(0, C)], sem_e[a])
        for a in (0, 1):
            ea_hbm, ii_hbm = arrs[a]
            pltpu.make_async_copy(ii_hbm.at[pl.ds(base_i, 1)],
                                  ibufs[2 * a], sem_i[2 * a]).wait()
            pltpu.make_async_copy(ea_hbm.at[pl.ds(base_e, C)],
                                  ebufs[a].at[pl.ds(0, C)], sem_e[a]).wait()
            pltpu.sync_copy(ebufs[a].at[pl.ds(0, C)],
                            acc.at[ibufs[2 * a].at[0]], add=True)

        plsc.subcore_barrier()

        @pl.loop(0, zc)
        def _(k):
            r = s * rows_sub + k * C
            pltpu.sync_copy(acc.at[pl.ds(r, C)], parts_hbm.at[c, pl.ds(r, C)])

    return sc_kernel(ea1, ea2, idx1, idx2, zrows)


def _tc_pool_body(parts_ref, ea0_ref, batch_ref, wt_ref, b_ref, out_ref,
                  acc_ref, cnt_ref, *, nb, g):
    i = pl.program_id(0)

    @pl.when(i == 0)
    def _():
        acc_ref[...] = jnp.zeros_like(acc_ref)
        cnt_ref[...] = jnp.zeros_like(cnt_ref)

    x = parts_ref[0] + parts_ref[1] + ea0_ref[...]
    h = jnp.dot(x, wt_ref[...], preferred_element_type=jnp.float32) + b_ref[...]
    h = jnp.where(h > 0, h, jnp.exp(jnp.minimum(h, 0.0)) - 1.0)

    bt = batch_ref[0, 0, :]
    gid = lax.broadcasted_iota(jnp.int32, (g, bt.shape[0]), 0)
    mask = (bt[None, :] == gid).astype(jnp.float32)
    acc_ref[...] += jnp.dot(mask, h, preferred_element_type=jnp.float32)
    cnt_ref[...] += jnp.broadcast_to(
        jnp.sum(mask, axis=1)[:, None], cnt_ref.shape)

    @pl.when(i == nb - 1)
    def _():
        out_ref[...] = acc_ref[...] / jnp.maximum(cnt_ref[...], 1.0)


def _tc_pool(parts, ea0p, batch3, wt, b2, bn, g):
    n, d = ea0p.shape
    nb = n // bn
    return pl.pallas_call(
        functools.partial(_tc_pool_body, nb=nb, g=g),
        grid=(nb,),
        in_specs=[
            pl.BlockSpec((NC, bn, d), lambda i: (0, i, 0)),
            pl.BlockSpec((bn, d), lambda i: (i, 0)),
            pl.BlockSpec((1, 1, bn), lambda i: (i, 0, 0)),
            pl.BlockSpec((d, d), lambda i: (0, 0)),
            pl.BlockSpec((1, d), lambda i: (0, 0)),
        ],
        out_specs=pl.BlockSpec((g, d), lambda i: (0, 0)),
        out_shape=jax.ShapeDtypeStruct((g, d), jnp.float32),
        scratch_shapes=[
            pltpu.VMEM((g, d), jnp.float32),
            pltpu.VMEM((g, 128), jnp.float32),
        ],
    )(parts, ea0p, batch3, wt, b2)


def kernel(edge_attr_0, edge_attr_1, edge_attr_2, edge_index_1, edge_index_2,
           num_nodes, batch, W, b):
    n, d = edge_attr_0.shape
    e = edge_attr_1.shape[0]
    g = 64
    bn = 2000
    n_pad = 10240

    idx1 = edge_index_1[0].astype(jnp.int32).reshape(e // C, C)
    idx2 = edge_index_2[0].astype(jnp.int32).reshape(e // C, C)
    zrows = jnp.zeros((C, d), jnp.float32)

    parts = _sc_scatter_parts(edge_attr_1, edge_attr_2, idx1, idx2, zrows, n_pad)

    batch3 = batch.astype(jnp.int32).reshape(n // bn, 1, bn)
    wt = W.T
    b2 = b.reshape(1, d)

    return _tc_pool(parts, edge_attr_0, batch3, wt, b2, bn, g)


# speedup vs baseline: 1.0787x; 1.0787x over previous
import functools

import jax
import jax.numpy as jnp
from jax import lax
from jax.experimental import pallas as pl
from jax.experimental.pallas import tpu as pltpu
from jax.experimental.pallas import tpu_sc as plsc

NC = 2
NS = 16
NW = NC * NS

C = 80


def _sc_scatter_parts(ea1, ea2, idx1, idx2, zrows, n_pad):
    e, d = ea1.shape
    ew = e // NW
    t = ew // C
    rows_sub = n_pad // NS
    zc = rows_sub // C

    mesh = plsc.VectorSubcoreMesh(core_axis_name="c", subcore_axis_name="s")

    @functools.partial(
        pl.kernel,
        out_type=jax.ShapeDtypeStruct((NC, n_pad, d), jnp.float32),
        mesh=mesh,
        scratch_types=[
            pltpu.VMEM_SHARED((n_pad, d), jnp.float32),
            [pltpu.VMEM((C, d), jnp.float32)] * 4,
            [pltpu.VMEM((1, C), jnp.int32)] * 4,
            [pltpu.SemaphoreType.DMA] * 4,
            [pltpu.SemaphoreType.DMA] * 4,
            [pltpu.SemaphoreType.DMA] * 4,
        ],
    )
    def sc_kernel(ea1_hbm, ea2_hbm, i1_hbm, i2_hbm, z_hbm, parts_hbm,
                  acc, ebufs, ibufs, sem_e, sem_i, sem_s):
        c = lax.axis_index("c")
        s = lax.axis_index("s")
        wid = c * NS + s
        base_e = wid * ew
        base_i = wid * t
        arrs = ((ea1_hbm, i1_hbm), (ea2_hbm, i2_hbm))

        def start_loads(a, b, q):
            ea_hbm, ii_hbm = arrs[a]
            k = 2 * a + b
            pltpu.async_copy(ii_hbm.at[pl.ds(base_i + q, 1)], ibufs[k], sem_i[k])
            pltpu.async_copy(ea_hbm.at[pl.ds(base_e + q * C, C)], ebufs[k], sem_e[k])

        def wait_loads(a, b):
            ea_hbm, ii_hbm = arrs[a]
            k = 2 * a + b
            pltpu.make_async_copy(ii_hbm.at[pl.ds(base_i, 1)], ibufs[k], sem_i[k]).wait()
            pltpu.make_async_copy(ea_hbm.at[pl.ds(base_e, C)], ebufs[k], sem_e[k]).wait()

        def start_scatter(a, b):
            k = 2 * a + b
            pltpu.async_copy(ebufs[k], acc.at[ibufs[k].at[0]], sem_s[k], add=True)

        def wait_scatter(a, b):
            k = 2 * a + b
            pltpu.make_async_copy(ebufs[k], acc.at[ibufs[k].at[0]], sem_s[k]).wait()

        for b in (0, 1):
            for a in (0, 1):
                start_loads(a, b, b)

        @pl.loop(0, zc)
        def _(k):
            pltpu.sync_copy(z_hbm, acc.at[pl.ds(s * rows_sub + k * C, C)])

        plsc.subcore_barrier()

        @pl.loop(0, t - 1, step=2)
        def _(m):
            for b in (0, 1):
                q = m + b
                for a in (0, 1):
                    wait_loads(a, b)
                    start_scatter(a, b)
                    wait_scatter(a, b)

                    @pl.when(q + 2 < t)
                    def _():
                        start_loads(a, b, q + 2)

        for a in (0, 1):
            wait_loads(a, 0)
            start_scatter(a, 0)
            wait_scatter(a, 0)

        plsc.subcore_barrier()

        @pl.loop(0, zc)
        def _(k):
            r = s * rows_sub + k * C
            pltpu.sync_copy(acc.at[pl.ds(r, C)], parts_hbm.at[c, pl.ds(r, C)])

    return sc_kernel(ea1, ea2, idx1, idx2, zrows)


def _tc_pool_body(parts_ref, ea0_ref, batch_ref, wt_ref, b_ref, out_ref,
                  acc_ref, cnt_ref, *, nb, g):
    i = pl.program_id(0)

    @pl.when(i == 0)
    def _():
        acc_ref[...] = jnp.zeros_like(acc_ref)
        cnt_ref[...] = jnp.zeros_like(cnt_ref)

    x = parts_ref[0] + parts_ref[1] + ea0_ref[...]
    h = jnp.dot(x, wt_ref[...], preferred_element_type=jnp.float32) + b_ref[...]
    h = jnp.where(h > 0, h, jnp.exp(jnp.minimum(h, 0.0)) - 1.0)

    bt = batch_ref[0, 0, :]
    gid = lax.broadcasted_iota(jnp.int32, (g, bt.shape[0]), 0)
    mask = (bt[None, :] == gid).astype(jnp.float32)
    acc_ref[...] += jnp.dot(mask, h, preferred_element_type=jnp.float32)
    cnt_ref[...] += jnp.broadcast_to(
        jnp.sum(mask, axis=1)[:, None], cnt_ref.shape)

    @pl.when(i == nb - 1)
    def _():
        out_ref[...] = acc_ref[...] / jnp.maximum(cnt_ref[...], 1.0)


def _tc_pool(parts, ea0p, batch3, wt, b2, bn, g):
    n, d = ea0p.shape
    nb = n // bn
    return pl.pallas_call(
        functools.partial(_tc_pool_body, nb=nb, g=g),
        grid=(nb,),
        in_specs=[
            pl.BlockSpec((NC, bn, d), lambda i: (0, i, 0)),
            pl.BlockSpec((bn, d), lambda i: (i, 0)),
            pl.BlockSpec((1, 1, bn), lambda i: (i, 0, 0)),
            pl.BlockSpec((d, d), lambda i: (0, 0)),
            pl.BlockSpec((1, d), lambda i: (0, 0)),
        ],
        out_specs=pl.BlockSpec((g, d), lambda i: (0, 0)),
        out_shape=jax.ShapeDtypeStruct((g, d), jnp.float32),
        scratch_shapes=[
            pltpu.VMEM((g, d), jnp.float32),
            pltpu.VMEM((g, 128), jnp.float32),
        ],
    )(parts, ea0p, batch3, wt, b2)


def kernel(edge_attr_0, edge_attr_1, edge_attr_2, edge_index_1, edge_index_2,
           num_nodes, batch, W, b):
    n, d = edge_attr_0.shape
    e = edge_attr_1.shape[0]
    g = 64
    bn = 2000
    n_pad = 10240

    idx1 = edge_index_1[0].astype(jnp.int32).reshape(e // C, C)
    idx2 = edge_index_2[0].astype(jnp.int32).reshape(e // C, C)
    zrows = jnp.zeros((C, d), jnp.float32)

    parts = _sc_scatter_parts(edge_attr_1, edge_attr_2, idx1, idx2, zrows, n_pad)

    batch3 = batch.astype(jnp.int32).reshape(n // bn, 1, bn)
    wt = W.T
    b2 = b.reshape(1, d)

    return _tc_pool(parts, edge_attr_0, batch3, wt, b2, bn, g)


# speedup vs baseline: 1.1337x; 1.0510x over previous
import functools

import jax
import jax.numpy as jnp
from jax import lax
from jax.experimental import pallas as pl
from jax.experimental.pallas import tpu as pltpu
from jax.experimental.pallas import tpu_sc as plsc

NC = 2
NS = 16
NW = NC * NS

C = 80


def _sc_scatter_parts(ea1, ea2, idx1, idx2, zrows, n_pad):
    e, d = ea1.shape
    ew = e // NW
    t = ew // C
    rows_sub = n_pad // NS

    mesh = plsc.VectorSubcoreMesh(core_axis_name="c", subcore_axis_name="s")

    @functools.partial(
        pl.kernel,
        out_type=jax.ShapeDtypeStruct((NC, n_pad, d), jnp.float32),
        mesh=mesh,
        scratch_types=[
            pltpu.VMEM_SHARED((n_pad, d), jnp.float32),
            [pltpu.VMEM((C, d), jnp.float32)] * 4,
            [pltpu.VMEM((1, C), jnp.int32)] * 4,
            [pltpu.SemaphoreType.DMA] * 4,
            [pltpu.SemaphoreType.DMA] * 4,
            [pltpu.SemaphoreType.DMA] * 4,
        ],
    )
    def sc_kernel(ea1_hbm, ea2_hbm, i1_hbm, i2_hbm, z_hbm, parts_hbm,
                  acc, ebufs, ibufs, sem_e, sem_i, sem_s):
        c = lax.axis_index("c")
        s = lax.axis_index("s")
        wid = c * NS + s
        base_e = wid * ew
        base_i = wid * t
        arrs = ((ea1_hbm, i1_hbm), (ea2_hbm, i2_hbm))

        def start_loads(a, b, q):
            ea_hbm, ii_hbm = arrs[a]
            k = 2 * a + b
            pltpu.async_copy(ii_hbm.at[pl.ds(base_i + q, 1)], ibufs[k], sem_i[k])
            pltpu.async_copy(ea_hbm.at[pl.ds(base_e + q * C, C)], ebufs[k], sem_e[k])

        def wait_loads(a, b):
            ea_hbm, ii_hbm = arrs[a]
            k = 2 * a + b
            pltpu.make_async_copy(ii_hbm.at[pl.ds(base_i, 1)], ibufs[k], sem_i[k]).wait()
            pltpu.make_async_copy(ea_hbm.at[pl.ds(base_e, C)], ebufs[k], sem_e[k]).wait()

        def start_scatter(a, b):
            k = 2 * a + b
            pltpu.async_copy(ebufs[k], acc.at[ibufs[k].at[0]], sem_s[k], add=True)

        def wait_scatter(a, b):
            k = 2 * a + b
            pltpu.make_async_copy(ebufs[k], acc.at[ibufs[k].at[0]], sem_s[k]).wait()

        for b in (0, 1):
            for a in (0, 1):
                start_loads(a, b, b)

        pltpu.sync_copy(z_hbm, acc.at[pl.ds(s * rows_sub, rows_sub)])

        plsc.subcore_barrier()

        @pl.loop(0, t - 1, step=2)
        def _(m):
            for b in (0, 1):
                q = m + b
                for a in (0, 1):
                    wait_loads(a, b)
                    start_scatter(a, b)
                    wait_scatter(a, b)

                    @pl.when(q + 2 < t)
                    def _():
                        start_loads(a, b, q + 2)

        for a in (0, 1):
            wait_loads(a, 0)
            start_scatter(a, 0)
            wait_scatter(a, 0)

        plsc.subcore_barrier()

        r = s * rows_sub
        pltpu.sync_copy(acc.at[pl.ds(r, rows_sub)],
                        parts_hbm.at[c, pl.ds(r, rows_sub)])

    return sc_kernel(ea1, ea2, idx1, idx2, zrows)


def _tc_pool_body(parts_ref, ea0_ref, batch_ref, wt_ref, b_ref, out_ref,
                  acc_ref, cnt_ref, *, nb, g):
    i = pl.program_id(0)

    @pl.when(i == 0)
    def _():
        acc_ref[...] = jnp.zeros_like(acc_ref)
        cnt_ref[...] = jnp.zeros_like(cnt_ref)

    x = parts_ref[0] + parts_ref[1] + ea0_ref[...]
    h = jnp.dot(x, wt_ref[...], preferred_element_type=jnp.float32) + b_ref[...]
    h = jnp.where(h > 0, h, jnp.exp(jnp.minimum(h, 0.0)) - 1.0)

    bt = batch_ref[0, 0, :]
    gid = lax.broadcasted_iota(jnp.int32, (g, bt.shape[0]), 0)
    mask = (bt[None, :] == gid).astype(jnp.float32)
    acc_ref[...] += jnp.dot(mask, h, preferred_element_type=jnp.float32)
    cnt_ref[...] += jnp.broadcast_to(
        jnp.sum(mask, axis=1)[:, None], cnt_ref.shape)

    @pl.when(i == nb - 1)
    def _():
        out_ref[...] = acc_ref[...] / jnp.maximum(cnt_ref[...], 1.0)


def _tc_pool(parts, ea0p, batch3, wt, b2, bn, g):
    n, d = ea0p.shape
    nb = n // bn
    return pl.pallas_call(
        functools.partial(_tc_pool_body, nb=nb, g=g),
        grid=(nb,),
        in_specs=[
            pl.BlockSpec((NC, bn, d), lambda i: (0, i, 0)),
            pl.BlockSpec((bn, d), lambda i: (i, 0)),
            pl.BlockSpec((1, 1, bn), lambda i: (i, 0, 0)),
            pl.BlockSpec((d, d), lambda i: (0, 0)),
            pl.BlockSpec((1, d), lambda i: (0, 0)),
        ],
        out_specs=pl.BlockSpec((g, d), lambda i: (0, 0)),
        out_shape=jax.ShapeDtypeStruct((g, d), jnp.float32),
        scratch_shapes=[
            pltpu.VMEM((g, d), jnp.float32),
            pltpu.VMEM((g, 128), jnp.float32),
        ],
    )(parts, ea0p, batch3, wt, b2)


def kernel(edge_attr_0, edge_attr_1, edge_attr_2, edge_index_1, edge_index_2,
           num_nodes, batch, W, b):
    n, d = edge_attr_0.shape
    e = edge_attr_1.shape[0]
    g = 64
    bn = 2000
    n_pad = 10240

    idx1 = edge_index_1[0].astype(jnp.int32).reshape(e // C, C)
    idx2 = edge_index_2[0].astype(jnp.int32).reshape(e // C, C)
    zrows = jnp.zeros((n_pad // NS, d), jnp.float32)

    parts = _sc_scatter_parts(edge_attr_1, edge_attr_2, idx1, idx2, zrows, n_pad)

    batch3 = batch.astype(jnp.int32).reshape(n // bn, 1, bn)
    wt = W.T
    b2 = b.reshape(1, d)

    return _tc_pool(parts, edge_attr_0, batch3, wt, b2, bn, g)
